# Initial kernel scaffold; baseline (speedup 1.0000x reference)
import jax
import jax.numpy as jnp
from jax.experimental import pallas as pl


def kernel(x, edge_index, W1, b1, Wm, bm, W2, b2):
    raise NotImplementedError("write your pallas kernel here")


# speedup vs baseline: 8.5865x; 8.5865x over previous
import functools

import jax
import jax.numpy as jnp
from jax import lax
from jax.experimental import pallas as pl
from jax.experimental.pallas import tpu as pltpu
from jax.experimental.pallas import tpu_sc as plsc

N = 10000
E = 320000
NC, NS = 2, 16
NW = NC * NS
EPW = E // NW
CH = 80
NCHUNK = EPW // CH
NP = 10240
RPT = NP // NS
ZR = 128
DDEG = 16

_MESH = plsc.VectorSubcoreMesh(
    core_axis_name="c", subcore_axis_name="s", num_cores=NC, num_subcores=NS)


def _zero_acc_slice(D, s, acc, zbuf):
    def zrow(i, carry):
        for j in range(D // 16):
            zbuf[i, pl.ds(j * 16, 16)] = jnp.zeros((16,), jnp.float32)
        return carry
    lax.fori_loop(0, ZR, zrow, 0)
    for k in range(RPT // ZR):
        pltpu.sync_copy(zbuf, acc.at[pl.ds(s * RPT + k * ZR, ZR)])


def _hop_body(D, u, row, col, out, acc, rbuf, cbuf, xbuf, zbuf, gsem):
    c = lax.axis_index("c")
    s = lax.axis_index("s")
    wid = c * NS + s
    _zero_acc_slice(D, s, acc, zbuf)
    plsc.subcore_barrier()

    def step(i, carry):
        base = wid * EPW + i * CH
        pltpu.sync_copy(row.at[pl.ds(base, CH)], rbuf.at[0])
        pltpu.sync_copy(col.at[pl.ds(base, CH)], cbuf.at[0])
        pltpu.async_copy(u.at[rbuf.at[0]], xbuf.at[0], gsem).wait()
        pltpu.sync_copy(xbuf.at[0], acc.at[cbuf.at[0]], add=True)
        return carry
    lax.fori_loop(0, NCHUNK, step, 0)

    plsc.subcore_barrier()
    pltpu.sync_copy(acc.at[pl.ds(s * RPT, RPT)],
                    out.at[pl.ds(c * NP + s * RPT, RPT)])


def _make_hop(D):
    return pl.kernel(
        functools.partial(_hop_body, D),
        out_type=jax.ShapeDtypeStruct((2 * NP, D), jnp.float32),
        mesh=_MESH,
        scratch_types=[
            pltpu.VMEM_SHARED((NP, D), jnp.float32),
            pltpu.VMEM((1, CH), jnp.int32),
            pltpu.VMEM((1, CH), jnp.int32),
            pltpu.VMEM((1, CH, D), jnp.float32),
            pltpu.VMEM((ZR, D), jnp.float32),
            pltpu.SemaphoreType.DMA,
        ],
        name=f"sgc_hop{D}",
    )


def _deg_body(col, out, acc, cbuf, xbuf, zbuf):
    c = lax.axis_index("c")
    s = lax.axis_index("s")
    wid = c * NS + s
    _zero_acc_slice(DDEG, s, acc, zbuf)

    def orow(i, carry):
        xbuf[i, :] = jnp.ones((16,), jnp.float32)
        return carry
    lax.fori_loop(0, CH, orow, 0)
    plsc.subcore_barrier()

    def step(i, carry):
        base = wid * EPW + i * CH
        pltpu.sync_copy(col.at[pl.ds(base, CH)], cbuf.at[0])
        pltpu.sync_copy(xbuf, acc.at[cbuf.at[0]], add=True)
        return carry
    lax.fori_loop(0, NCHUNK, step, 0)

    plsc.subcore_barrier()
    pltpu.sync_copy(acc.at[pl.ds(s * RPT, RPT)],
                    out.at[pl.ds(c * NP + s * RPT, RPT)])


_deg_hop = pl.kernel(
    _deg_body,
    out_type=jax.ShapeDtypeStruct((2 * NP, DDEG), jnp.float32),
    mesh=_MESH,
    scratch_types=[
        pltpu.VMEM_SHARED((NP, DDEG), jnp.float32),
        pltpu.VMEM((1, CH), jnp.int32),
        pltpu.VMEM((CH, DDEG), jnp.float32),
        pltpu.VMEM((ZR, DDEG), jnp.float32),
    ],
    name="sgc_deg",
)

_hop128 = _make_hop(128)

BR = 1000
_GRID = (N // BR,)


def _rows_spec(d):
    return pl.BlockSpec((BR, d), lambda i: (i, 0))


def _full_spec(shape):
    return pl.BlockSpec(shape, lambda i: tuple(0 for _ in shape))


def _t0_body(d0, d1, x, dis, dis2, u):
    deg = d0[:, 0:1] + d1[:, 0:1] + 1.0
    di = lax.rsqrt(deg)
    dis[...] = di
    dis2[...] = di * di
    u[...] = x[...] * di


def _t0(d0, d1, x):
    return pl.pallas_call(
        _t0_body,
        grid=_GRID,
        in_specs=[_rows_spec(DDEG), _rows_spec(DDEG), _rows_spec(128)],
        out_specs=[_rows_spec(1), _rows_spec(1), _rows_spec(128)],
        out_shape=[jax.ShapeDtypeStruct((N, 1), jnp.float32),
                   jax.ShapeDtypeStruct((N, 1), jnp.float32),
                   jax.ShapeDtypeStruct((N, 128), jnp.float32)],
    )(d0, d1, x)


def _mid_body(p0, p1, u, dis2, w):
    w[...] = dis2[...] * (p0[...] + p1[...] + u[...])


def _mid(p0, p1, u, dis2, d):
    return pl.pallas_call(
        _mid_body,
        grid=_GRID,
        in_specs=[_rows_spec(d), _rows_spec(d), _rows_spec(d), _rows_spec(1)],
        out_specs=_rows_spec(d),
        out_shape=jax.ShapeDtypeStruct((N, d), jnp.float32),
    )(p0, p1, u, dis2)


def _post1_body(q0, q1, w, dis, W, b, u2):
    z = dis[...] * (q0[...] + q1[...] + w[...])
    h = jax.nn.relu(jnp.dot(z, W[...], preferred_element_type=jnp.float32)
                    + b[...])
    u2[...] = dis[...] * h


def _post1(q0, q1, w, dis, W, b):
    return pl.pallas_call(
        _post1_body,
        grid=_GRID,
        in_specs=[_rows_spec(128), _rows_spec(128), _rows_spec(128),
                  _rows_spec(1), _full_spec((128, 128)), _full_spec((1, 128))],
        out_specs=_rows_spec(128),
        out_shape=jax.ShapeDtypeStruct((N, 128), jnp.float32),
    )(q0, q1, w, dis, W, b)


def _post2_body(q0, q1, w, dis, Wm, bm, W2p, u3):
    z = dis[...] * (q0[...] + q1[...] + w[...])
    h = jax.nn.relu(jnp.dot(z, Wm[...], preferred_element_type=jnp.float32)
                    + bm[...])
    g = jnp.dot(h, W2p[...], preferred_element_type=jnp.float32)
    u3[...] = dis[...] * g


def _post2(q0, q1, w, dis, Wm, bm, W2p):
    return pl.pallas_call(
        _post2_body,
        grid=_GRID,
        in_specs=[_rows_spec(128), _rows_spec(128), _rows_spec(128),
                  _rows_spec(1), _full_spec((128, 128)), _full_spec((1, 128)),
                  _full_spec((128, 128))],
        out_specs=_rows_spec(128),
        out_shape=jax.ShapeDtypeStruct((N, 128), jnp.float32),
    )(q0, q1, w, dis, Wm, bm, W2p)


def _final_body(q0, q1, w, dis, b2, out):
    t = (dis[...] * (q0[...] + q1[...] + w[...]))[:, :64] + b2[...]
    m = jnp.max(t, axis=1, keepdims=True)
    lse = jnp.log(jnp.sum(jnp.exp(t - m), axis=1, keepdims=True)) + m
    out[...] = t - lse


def _final(q0, q1, w, dis, b2):
    return pl.pallas_call(
        _final_body,
        grid=_GRID,
        in_specs=[_rows_spec(128), _rows_spec(128), _rows_spec(128),
                  _rows_spec(1), _full_spec((1, 64))],
        out_specs=_rows_spec(64),
        out_shape=jax.ShapeDtypeStruct((N, 64), jnp.float32),
    )(q0, q1, w, dis, b2)


def kernel(x, edge_index, W1, b1, Wm, bm, W2, b2):
    row = edge_index[0].astype(jnp.int32)
    col = edge_index[1].astype(jnp.int32)
    b1r = b1.reshape(1, -1)
    bmr = bm.reshape(1, -1)
    b2r = b2.reshape(1, -1)
    W2p = jnp.pad(W2, ((0, 0), (0, 64)))

    degp = _deg_hop(col)
    dis, dis2, u = _t0(degp[:N], degp[NP:NP + N], x)

    p = _hop128(u, row, col)
    w = _mid(p[:N], p[NP:NP + N], u, dis2, 128)
    q = _hop128(w, row, col)
    u = _post1(q[:N], q[NP:NP + N], w, dis, W1, b1r)

    p = _hop128(u, row, col)
    w = _mid(p[:N], p[NP:NP + N], u, dis2, 128)
    q = _hop128(w, row, col)
    u = _post2(q[:N], q[NP:NP + N], w, dis, Wm, bmr, W2p)

    p = _hop128(u, row, col)
    w = _mid(p[:N], p[NP:NP + N], u, dis2, 128)
    q = _hop128(w, row, col)
    return _final(q[:N], q[NP:NP + N], w, dis, b2r)
